# baseline (speedup 1.0000x reference)
import functools

import jax
import jax.numpy as jnp
from jax import lax
from jax.experimental import pallas as pl
from jax.experimental.pallas import tpu as pltpu
from jax.experimental.pallas import tpu_sc as plsc

NC = 2
NS = 16
LANES = 16
K = 128
NB = 2
SP = 4
BM = 1024


def _cdiv(a, b):
    return (a + b - 1) // b


def _zero_shared(rows_list, table_sh, sid, Np, nf):
    zero = jnp.zeros((LANES,), jnp.float32)

    for rows_v in rows_list:
        def zrow(k, _):
            for f in range(nf):
                rows_v[k, pl.ds(f * LANES, LANES)] = zero
            return _

        lax.fori_loop(0, K, zrow, None)
    for t in range(Np // NS // K):
        pltpu.sync_copy(rows_list[0],
                        table_sh.at[pl.ds(sid * (Np // NS) + t * K, K)])


def _make_agg_kernel(Np, Ep, F, gather, sp=1):
    cpt = Ep // (NC * NS * K)
    nj = cpt // NB
    nf = F // LANES
    ow = F
    mesh = plsc.VectorSubcoreMesh(
        core_axis_name="c", subcore_axis_name="s", num_cores=NC, num_subcores=NS)

    scratch = (
        [pltpu.VMEM((K,), jnp.int32) for _ in range(NB)]
        + [pltpu.VMEM((K,), jnp.int32) for _ in range(NB)]
        + [pltpu.VMEM((K,), jnp.float32) for _ in range(NB)]
        + [pltpu.VMEM((K, F), jnp.float32) for _ in range(NB)]
        + [pltpu.VMEM_SHARED((Np, F), jnp.float32)]
        + [pltpu.SemaphoreType.DMA for _ in range(3 * NB)]
    )

    @functools.partial(
        pl.kernel,
        out_type=jax.ShapeDtypeStruct((NC, Np, ow), jnp.float32),
        mesh=mesh,
        scratch_types=scratch,
    )
    def agg_kernel(y_hbm, src_hbm, dst_hbm, ew_hbm, out_hbm, *scr):
        s_idx = scr[0:NB]
        d_idx = scr[NB:2 * NB]
        w_v = scr[2 * NB:3 * NB]
        rows = scr[3 * NB:4 * NB]
        acc_sh = scr[4 * NB]
        isem = scr[4 * NB + 1:4 * NB + 1 + NB]
        gsem = scr[4 * NB + 1 + NB:4 * NB + 1 + 2 * NB]
        ssem = scr[4 * NB + 1 + 2 * NB:4 * NB + 1 + 3 * NB]

        cid = lax.axis_index("c")
        sid = lax.axis_index("s")
        wid = cid * NS + sid
        base = wid * cpt

        _zero_shared(rows, acc_sh, sid, Np, nf)
        plsc.subcore_barrier()

        def fetch(b, j):
            off = (base + j) * K
            if gather:
                pltpu.async_copy(src_hbm.at[pl.ds(off, K)], s_idx[b], isem[b])
            pltpu.async_copy(dst_hbm.at[pl.ds(off, K)], d_idx[b], isem[b])
            pltpu.async_copy(ew_hbm.at[pl.ds(off, K)], w_v[b], isem[b])
            if gather:
                pltpu.make_async_copy(src_hbm.at[pl.ds(off, K)], s_idx[b],
                                      isem[b]).wait()
            pltpu.make_async_copy(dst_hbm.at[pl.ds(off, K)], d_idx[b],
                                  isem[b]).wait()
            pltpu.make_async_copy(ew_hbm.at[pl.ds(off, K)], w_v[b],
                                  isem[b]).wait()
            if gather:
                if sp > 1:
                    for i in range(K // LANES):
                        sl = pl.ds(i * LANES, LANES)
                        s_idx[b][sl] = s_idx[b][sl] * sp
                pltpu.async_copy(y_hbm.at[s_idx[b]], rows[b], gsem[b])

        def wait_gather(b):
            if gather:
                pltpu.make_async_copy(y_hbm.at[s_idx[b]], rows[b],
                                      gsem[b]).wait()

        def scale(b):
            def grp(g, _):
                w16 = w_v[b][pl.ds(g * LANES, LANES)]
                for l in range(LANES):
                    k = g * LANES + l
                    w = jnp.full((LANES,), w16[l], jnp.float32)
                    if gather:
                        for f in range(nf):
                            sl = pl.ds(f * LANES, LANES)
                            rows[b][k, sl] = rows[b][k, sl] * w
                    else:
                        rows[b][k, pl.ds(0, LANES)] = w
                return _

            lax.fori_loop(0, K // LANES, grp, None)

        def start_scatter(b):
            pltpu.async_copy(rows[b], acc_sh.at[d_idx[b]], ssem[b], add=True)

        def wait_scatter(b):
            pltpu.make_async_copy(rows[b], acc_sh.at[d_idx[b]], ssem[b]).wait()

        for b in range(NB):
            fetch(b, b)

        def body(jj, _):
            for b in range(NB):
                wait_gather(b)
                scale(b)
                start_scatter(b)
            for b in range(NB):
                wait_scatter(b)
                fetch(b, jj * NB + b)
            return _

        lax.fori_loop(1, nj, body, None)

        for b in range(NB):
            wait_gather(b)
            scale(b)
            start_scatter(b)
        for b in range(NB):
            wait_scatter(b)

        plsc.subcore_barrier()
        for t in range(Np // NS // K):
            r0 = sid * (Np // NS) + t * K
            if ow == F:
                pltpu.sync_copy(acc_sh.at[pl.ds(r0, K)],
                                out_hbm.at[cid, pl.ds(r0, K)])
            else:
                pltpu.sync_copy(acc_sh.at[pl.ds(r0, K), pl.ds(0, ow)],
                                out_hbm.at[cid, pl.ds(r0, K)])

    return agg_kernel


def _dinv_from_deg(dp):
    deg = dp[0, :, 0] + dp[1, :, 0] + 1.0
    return jnp.where(deg > 0, lax.rsqrt(jnp.maximum(deg, 1e-12)), 0.0)


def _tc_first(x_ref, w_ref, dp_ref, y_ref, y4_ref):
    dinv = _dinv_from_deg(dp_ref[...])
    xw = jnp.dot(x_ref[...], w_ref[...], preferred_element_type=jnp.float32)
    y = dinv[:, None] * xw
    y_ref[...] = y
    y4_ref[:, 0:y.shape[1]] = y


def _tc_mid(a0_ref, a1_ref, y1_ref, dp_ref, b_ref, w_ref, y2_ref, y4_ref):
    dinv = _dinv_from_deg(dp_ref[...])
    pre = dinv[:, None] * (a0_ref[...] + a1_ref[...] + y1_ref[...]) + b_ref[...]
    h = jnp.where(pre >= 0, pre, 0.01 * pre)
    hw = jnp.dot(h, w_ref[...], preferred_element_type=jnp.float32)
    y2 = dinv[:, None] * hw
    y2_ref[...] = y2
    y4_ref[:, 0:y2.shape[1]] = y2


def _tc_last(a0_ref, a1_ref, y2_ref, dp_ref, b_ref, out_ref):
    dinv = _dinv_from_deg(dp_ref[...])
    out_ref[...] = dinv[:, None] * (a0_ref[...] + a1_ref[...] + y2_ref[...]) + b_ref[...]


def _row_spec(F):
    return pl.BlockSpec((BM, F), lambda i: (i, 0))


def _tc_call(body, Np, F, specs, spread_out=False):
    if spread_out:
        out_specs = [_row_spec(F), _row_spec(SP * F)]
        out_shape = [jax.ShapeDtypeStruct((Np, F), jnp.float32),
                     jax.ShapeDtypeStruct((Np, SP * F), jnp.float32)]
    else:
        out_specs = _row_spec(F)
        out_shape = jax.ShapeDtypeStruct((Np, F), jnp.float32)
    return pl.pallas_call(
        body,
        grid=(Np // BM,),
        in_specs=specs,
        out_specs=out_specs,
        out_shape=out_shape,
    )


def kernel(x, edge_index, edge_weight, W1, b1, W2, b2):
    N, F = x.shape
    E = edge_index.shape[1]
    Np = _cdiv(N, NS * K) * NS * K
    Ep = _cdiv(E, NC * NS * K * NB) * NC * NS * K * NB

    src = edge_index[0]
    dst = edge_index[1]
    if Ep != E:
        pad = Ep - E
        spread = (jnp.arange(pad, dtype=jnp.int32) * 97) % N
        src = jnp.concatenate([src, spread])
        dst = jnp.concatenate([dst, spread])
        edge_weight = jnp.pad(edge_weight, (0, pad))
    xp = jnp.pad(x, ((0, Np - N), (0, 0))) if Np != N else x
    b1r = b1.reshape(1, F)
    b2r = b2.reshape(1, F)

    deg_k = _make_agg_kernel(Np, Ep, F, gather=False)
    agg_k = _make_agg_kernel(Np, Ep, F, gather=True, sp=SP)

    degp = deg_k(xp, src, dst, edge_weight)

    dp_spec = pl.BlockSpec((NC, BM, F), lambda i: (0, i, 0))
    w_spec = pl.BlockSpec((F, F), lambda i: (0, 0))
    b_spec = pl.BlockSpec((1, F), lambda i: (0, 0))
    row = _row_spec(F)

    y1, y14 = _tc_call(_tc_first, Np, F, [row, w_spec, dp_spec],
                       spread_out=True)(xp, W1, degp)

    acc1 = agg_k(y14.reshape(SP * Np, F), src, dst, edge_weight)

    y2, y24 = _tc_call(_tc_mid, Np, F, [row, row, row, dp_spec, b_spec, w_spec],
                       spread_out=True)(acc1[0], acc1[1], y1, degp, b1r, W2)

    acc2 = agg_k(y24.reshape(SP * Np, F), src, dst, edge_weight)

    out = _tc_call(_tc_last, Np, F, [row, row, row, dp_spec, b_spec])(
        acc2[0], acc2[1], y2, degp, b2r)

    return out[:N]


# speedup vs baseline: 1.2646x; 1.2646x over previous
import functools

import jax
import jax.numpy as jnp
from jax import lax
from jax.experimental import pallas as pl
from jax.experimental.pallas import tpu as pltpu
from jax.experimental.pallas import tpu_sc as plsc

NC = 2
NS = 16
LANES = 16
K = 128
NB = 2
BM = 1024


def _cdiv(a, b):
    return (a + b - 1) // b


def _zero_shared(rows_list, table_sh, sid, Np, nf):
    zero = jnp.zeros((LANES,), jnp.float32)

    for rows_v in rows_list:
        def zrow(k, _):
            for f in range(nf):
                rows_v[k, pl.ds(f * LANES, LANES)] = zero
            return _

        lax.fori_loop(0, K, zrow, None)
    for t in range(Np // NS // K):
        pltpu.sync_copy(rows_list[0],
                        table_sh.at[pl.ds(sid * (Np // NS) + t * K, K)])


def _make_deg_kernel(Np, Ep):
    Et = Ep // (NC * NS)
    NR = 5120
    assert Np % NR == 0
    rounds = Np // NR
    stride = NR + 1
    mesh = plsc.VectorSubcoreMesh(
        core_axis_name="c", subcore_axis_name="s", num_cores=NC, num_subcores=NS)

    @functools.partial(
        pl.kernel,
        out_type=jax.ShapeDtypeStruct((NC * NS, Np), jnp.float32),
        mesh=mesh,
        compiler_params=pltpu.CompilerParams(needs_layout_passes=False),
        scratch_types=[
            pltpu.VMEM((Et,), jnp.int32),
            pltpu.VMEM((Et,), jnp.float32),
            pltpu.VMEM((LANES * stride,), jnp.float32),
            pltpu.VMEM((NR,), jnp.float32),
            pltpu.SemaphoreType.DMA,
        ],
    )
    def deg_kernel(dst_hbm, ew_hbm, out_hbm, d_all, w_all, tbl, red, sem):
        cid = lax.axis_index("c")
        sid = lax.axis_index("s")
        wid = cid * NS + sid
        zero = jnp.zeros((LANES,), jnp.float32)
        lanes_base = lax.iota(jnp.int32, LANES) * stride

        pltpu.async_copy(dst_hbm.at[pl.ds(wid * Et, Et)], d_all, sem)
        pltpu.async_copy(ew_hbm.at[pl.ds(wid * Et, Et)], w_all, sem)

        def ztbl(i, _):
            tbl[pl.ds(i * LANES, LANES)] = zero
            return _

        lax.fori_loop(0, stride, ztbl, None)

        pltpu.make_async_copy(dst_hbm.at[pl.ds(wid * Et, Et)], d_all, sem).wait()
        pltpu.make_async_copy(ew_hbm.at[pl.ds(wid * Et, Et)], w_all, sem).wait()

        for r in range(rounds):
            lo = r * NR

            def edge(g, _):
                sl = pl.ds(g * LANES, LANES)
                off = d_all[sl] - lo
                outside = ((off >> 31) & 1) | (((NR - 1 - off) >> 31) & 1)
                idx = jnp.minimum(jnp.maximum(off, 0), NR - 1) + lanes_base
                wm = w_all[sl] * (1 - outside).astype(jnp.float32)
                plsc.addupdate_scatter(tbl, [idx], wm)
                return _

            lax.fori_loop(0, Et // LANES, edge, None)

            def reduce(c, _):
                sl = pl.ds(c * LANES, LANES)
                acc = tbl[pl.ds(c * LANES, LANES)]
                if r + 1 < rounds:
                    tbl[pl.ds(c * LANES, LANES)] = zero
                for l in range(1, LANES):
                    sll = pl.ds(l * stride + c * LANES, LANES)
                    acc = acc + tbl[sll]
                    if r + 1 < rounds:
                        tbl[sll] = zero
                red[sl] = acc
                return _

            lax.fori_loop(0, NR // LANES, reduce, None)
            pltpu.sync_copy(red, out_hbm.at[wid, pl.ds(lo, NR)])

    return deg_kernel


def _make_agg_kernel(Np, Ep, F, gather):
    cpt = Ep // (NC * NS * K)
    nj = cpt // NB
    nf = F // LANES
    mesh = plsc.VectorSubcoreMesh(
        core_axis_name="c", subcore_axis_name="s", num_cores=NC, num_subcores=NS)

    scratch = (
        [pltpu.VMEM((K,), jnp.int32) for _ in range(NB)]
        + [pltpu.VMEM((K,), jnp.int32) for _ in range(NB)]
        + [pltpu.VMEM((K,), jnp.float32) for _ in range(NB)]
        + [pltpu.VMEM((K, F), jnp.float32) for _ in range(NB)]
        + [pltpu.VMEM_SHARED((Np, F), jnp.float32)]
        + [pltpu.SemaphoreType.DMA for _ in range(3 * NB)]
    )

    @functools.partial(
        pl.kernel,
        out_type=jax.ShapeDtypeStruct((NC, Np, F), jnp.float32),
        mesh=mesh,
        scratch_types=scratch,
    )
    def agg_kernel(y_hbm, src_hbm, dst_hbm, ew_hbm, out_hbm, *scr):
        s_idx = scr[0:NB]
        d_idx = scr[NB:2 * NB]
        w_v = scr[2 * NB:3 * NB]
        rows = scr[3 * NB:4 * NB]
        acc_sh = scr[4 * NB]
        isem = scr[4 * NB + 1:4 * NB + 1 + NB]
        gsem = scr[4 * NB + 1 + NB:4 * NB + 1 + 2 * NB]
        ssem = scr[4 * NB + 1 + 2 * NB:4 * NB + 1 + 3 * NB]

        cid = lax.axis_index("c")
        sid = lax.axis_index("s")
        wid = cid * NS + sid
        base = wid * cpt

        _zero_shared(rows, acc_sh, sid, Np, nf)
        plsc.subcore_barrier()

        def fetch(b, j):
            off = (base + j) * K
            if gather:
                pltpu.async_copy(src_hbm.at[pl.ds(off, K)], s_idx[b], isem[b])
            pltpu.async_copy(dst_hbm.at[pl.ds(off, K)], d_idx[b], isem[b])
            pltpu.async_copy(ew_hbm.at[pl.ds(off, K)], w_v[b], isem[b])
            if gather:
                pltpu.make_async_copy(src_hbm.at[pl.ds(off, K)], s_idx[b],
                                      isem[b]).wait()
            pltpu.make_async_copy(dst_hbm.at[pl.ds(off, K)], d_idx[b],
                                  isem[b]).wait()
            pltpu.make_async_copy(ew_hbm.at[pl.ds(off, K)], w_v[b],
                                  isem[b]).wait()
            if gather:
                pltpu.async_copy(y_hbm.at[s_idx[b]], rows[b], gsem[b])

        def wait_gather(b):
            if gather:
                pltpu.make_async_copy(y_hbm.at[s_idx[b]], rows[b],
                                      gsem[b]).wait()

        def scale(b):
            def grp(g, _):
                w16 = w_v[b][pl.ds(g * LANES, LANES)]
                for l in range(LANES):
                    k = g * LANES + l
                    w = jnp.full((LANES,), w16[l], jnp.float32)
                    if gather:
                        for f in range(nf):
                            sl = pl.ds(f * LANES, LANES)
                            rows[b][k, sl] = rows[b][k, sl] * w
                    else:
                        rows[b][k, pl.ds(0, LANES)] = w
                return _

            lax.fori_loop(0, K // LANES, grp, None)

        def start_scatter(b):
            pltpu.async_copy(rows[b], acc_sh.at[d_idx[b]], ssem[b], add=True)

        def wait_scatter(b):
            pltpu.make_async_copy(rows[b], acc_sh.at[d_idx[b]], ssem[b]).wait()

        for b in range(NB):
            fetch(b, b)

        def body(jj, _):
            for b in range(NB):
                wait_gather(b)
                scale(b)
                start_scatter(b)
            for b in range(NB):
                wait_scatter(b)
                fetch(b, jj * NB + b)
            return _

        lax.fori_loop(1, nj, body, None)

        for b in range(NB):
            wait_gather(b)
            scale(b)
            start_scatter(b)
        for b in range(NB):
            wait_scatter(b)

        plsc.subcore_barrier()
        for t in range(Np // NS // K):
            r0 = sid * (Np // NS) + t * K
            pltpu.sync_copy(acc_sh.at[pl.ds(r0, K)], out_hbm.at[cid, pl.ds(r0, K)])

    return agg_kernel


def _dinv_from_deg(dp):
    deg = jnp.sum(dp, axis=0) + 1.0
    return jnp.where(deg > 0, lax.rsqrt(jnp.maximum(deg, 1e-12)), 0.0)


def _tc_first(x_ref, w_ref, dp_ref, y_ref):
    dinv = _dinv_from_deg(dp_ref[...])
    xw = jnp.dot(x_ref[...], w_ref[...], preferred_element_type=jnp.float32)
    y_ref[...] = dinv[:, None] * xw


def _tc_mid(a0_ref, a1_ref, y1_ref, dp_ref, b_ref, w_ref, y2_ref):
    dinv = _dinv_from_deg(dp_ref[...])
    pre = dinv[:, None] * (a0_ref[...] + a1_ref[...] + y1_ref[...]) + b_ref[...]
    h = jnp.where(pre >= 0, pre, 0.01 * pre)
    hw = jnp.dot(h, w_ref[...], preferred_element_type=jnp.float32)
    y2_ref[...] = dinv[:, None] * hw


def _tc_last(a0_ref, a1_ref, y2_ref, dp_ref, b_ref, out_ref):
    dinv = _dinv_from_deg(dp_ref[...])
    out_ref[...] = dinv[:, None] * (a0_ref[...] + a1_ref[...] + y2_ref[...]) + b_ref[...]


def _row_spec(F):
    return pl.BlockSpec((BM, F), lambda i: (i, 0))


def _tc_call(body, Np, F, specs):
    return pl.pallas_call(
        body,
        grid=(Np // BM,),
        in_specs=specs,
        out_specs=_row_spec(F),
        out_shape=jax.ShapeDtypeStruct((Np, F), jnp.float32),
    )


def kernel(x, edge_index, edge_weight, W1, b1, W2, b2):
    N, F = x.shape
    E = edge_index.shape[1]
    Np = _cdiv(N, NS * K) * NS * K
    Ep = _cdiv(E, NC * NS * K * NB) * NC * NS * K * NB

    src = edge_index[0]
    dst = edge_index[1]
    if Ep != E:
        pad = Ep - E
        spread = (jnp.arange(pad, dtype=jnp.int32) * 97) % N
        src = jnp.concatenate([src, spread])
        dst = jnp.concatenate([dst, spread])
        edge_weight = jnp.pad(edge_weight, (0, pad))
    xp = jnp.pad(x, ((0, Np - N), (0, 0))) if Np != N else x
    b1r = b1.reshape(1, F)
    b2r = b2.reshape(1, F)

    deg_k = _make_deg_kernel(Np, Ep)
    agg_k = _make_agg_kernel(Np, Ep, F, gather=True)

    degp = deg_k(dst, edge_weight)

    dp_spec = pl.BlockSpec((NC * NS, BM), lambda i: (0, i))
    w_spec = pl.BlockSpec((F, F), lambda i: (0, 0))
    b_spec = pl.BlockSpec((1, F), lambda i: (0, 0))
    row = _row_spec(F)

    y1 = _tc_call(_tc_first, Np, F, [row, w_spec, dp_spec])(xp, W1, degp)

    acc1 = agg_k(y1, src, dst, edge_weight)

    y2 = _tc_call(_tc_mid, Np, F, [row, row, row, dp_spec, b_spec, w_spec])(
        acc1[0], acc1[1], y1, degp, b1r, W2)

    acc2 = agg_k(y2, src, dst, edge_weight)

    out = _tc_call(_tc_last, Np, F, [row, row, row, dp_spec, b_spec])(
        acc2[0], acc2[1], y2, degp, b2r)

    return out[:N]


# speedup vs baseline: 1.2826x; 1.0142x over previous
import functools

import jax
import jax.numpy as jnp
from jax import lax
from jax.experimental import pallas as pl
from jax.experimental.pallas import tpu as pltpu
from jax.experimental.pallas import tpu_sc as plsc

NC = 2
NS = 16
LANES = 16
K = 64
NB = 4
BM = 1024


def _cdiv(a, b):
    return (a + b - 1) // b


def _zero_shared(rows_list, table_sh, sid, Np, nf):
    zero = jnp.zeros((LANES,), jnp.float32)

    for rows_v in rows_list:
        def zrow(k, _):
            for f in range(nf):
                rows_v[k, pl.ds(f * LANES, LANES)] = zero
            return _

        lax.fori_loop(0, K, zrow, None)
    for t in range(Np // NS // K):
        pltpu.sync_copy(rows_list[0],
                        table_sh.at[pl.ds(sid * (Np // NS) + t * K, K)])


def _make_deg_kernel(Np, Ep):
    Et = Ep // (NC * NS)
    NR = 5120
    assert Np % NR == 0
    rounds = Np // NR
    stride = NR + 1
    mesh = plsc.VectorSubcoreMesh(
        core_axis_name="c", subcore_axis_name="s", num_cores=NC, num_subcores=NS)

    @functools.partial(
        pl.kernel,
        out_type=jax.ShapeDtypeStruct((NC * NS, Np), jnp.float32),
        mesh=mesh,
        compiler_params=pltpu.CompilerParams(needs_layout_passes=False),
        scratch_types=[
            pltpu.VMEM((Et,), jnp.int32),
            pltpu.VMEM((Et,), jnp.float32),
            pltpu.VMEM((LANES * stride,), jnp.float32),
            pltpu.VMEM((NR,), jnp.float32),
            pltpu.SemaphoreType.DMA,
        ],
    )
    def deg_kernel(dst_hbm, ew_hbm, out_hbm, d_all, w_all, tbl, red, sem):
        cid = lax.axis_index("c")
        sid = lax.axis_index("s")
        wid = cid * NS + sid
        zero = jnp.zeros((LANES,), jnp.float32)
        lanes_base = lax.iota(jnp.int32, LANES) * stride

        pltpu.async_copy(dst_hbm.at[pl.ds(wid * Et, Et)], d_all, sem)
        pltpu.async_copy(ew_hbm.at[pl.ds(wid * Et, Et)], w_all, sem)

        def ztbl(i, _):
            tbl[pl.ds(i * LANES, LANES)] = zero
            return _

        lax.fori_loop(0, stride, ztbl, None)

        pltpu.make_async_copy(dst_hbm.at[pl.ds(wid * Et, Et)], d_all, sem).wait()
        pltpu.make_async_copy(ew_hbm.at[pl.ds(wid * Et, Et)], w_all, sem).wait()

        for r in range(rounds):
            lo = r * NR

            def edge(g, _):
                sl = pl.ds(g * LANES, LANES)
                off = d_all[sl] - lo
                outside = ((off >> 31) & 1) | (((NR - 1 - off) >> 31) & 1)
                idx = jnp.minimum(jnp.maximum(off, 0), NR - 1) + lanes_base
                wm = w_all[sl] * (1 - outside).astype(jnp.float32)
                plsc.addupdate_scatter(tbl, [idx], wm)
                return _

            lax.fori_loop(0, Et // LANES, edge, None)

            def reduce(c, _):
                sl = pl.ds(c * LANES, LANES)
                acc = tbl[pl.ds(c * LANES, LANES)]
                if r + 1 < rounds:
                    tbl[pl.ds(c * LANES, LANES)] = zero
                for l in range(1, LANES):
                    sll = pl.ds(l * stride + c * LANES, LANES)
                    acc = acc + tbl[sll]
                    if r + 1 < rounds:
                        tbl[sll] = zero
                red[sl] = acc
                return _

            lax.fori_loop(0, NR // LANES, reduce, None)
            pltpu.sync_copy(red, out_hbm.at[wid, pl.ds(lo, NR)])

    return deg_kernel


def _make_agg_kernel(Np, Ep, F, gather):
    cpt = Ep // (NC * NS * K)
    nj = cpt // NB
    nf = F // LANES
    mesh = plsc.VectorSubcoreMesh(
        core_axis_name="c", subcore_axis_name="s", num_cores=NC, num_subcores=NS)

    scratch = (
        [pltpu.VMEM((K,), jnp.int32) for _ in range(NB)]
        + [pltpu.VMEM((K,), jnp.int32) for _ in range(NB)]
        + [pltpu.VMEM((K,), jnp.float32) for _ in range(NB)]
        + [pltpu.VMEM((K, F), jnp.float32) for _ in range(NB)]
        + [pltpu.VMEM_SHARED((Np, F), jnp.float32)]
        + [pltpu.SemaphoreType.DMA for _ in range(3 * NB)]
    )

    @functools.partial(
        pl.kernel,
        out_type=jax.ShapeDtypeStruct((NC, Np, F), jnp.float32),
        mesh=mesh,
        scratch_types=scratch,
    )
    def agg_kernel(y_hbm, src_hbm, dst_hbm, ew_hbm, out_hbm, *scr):
        s_idx = scr[0:NB]
        d_idx = scr[NB:2 * NB]
        w_v = scr[2 * NB:3 * NB]
        rows = scr[3 * NB:4 * NB]
        acc_sh = scr[4 * NB]
        isem = scr[4 * NB + 1:4 * NB + 1 + NB]
        gsem = scr[4 * NB + 1 + NB:4 * NB + 1 + 2 * NB]
        ssem = scr[4 * NB + 1 + 2 * NB:4 * NB + 1 + 3 * NB]

        cid = lax.axis_index("c")
        sid = lax.axis_index("s")
        wid = cid * NS + sid
        base = wid * cpt

        _zero_shared(rows, acc_sh, sid, Np, nf)
        plsc.subcore_barrier()

        def fetch(b, j):
            off = (base + j) * K
            if gather:
                pltpu.async_copy(src_hbm.at[pl.ds(off, K)], s_idx[b], isem[b])
            pltpu.async_copy(dst_hbm.at[pl.ds(off, K)], d_idx[b], isem[b])
            pltpu.async_copy(ew_hbm.at[pl.ds(off, K)], w_v[b], isem[b])
            if gather:
                pltpu.make_async_copy(src_hbm.at[pl.ds(off, K)], s_idx[b],
                                      isem[b]).wait()
            pltpu.make_async_copy(dst_hbm.at[pl.ds(off, K)], d_idx[b],
                                  isem[b]).wait()
            pltpu.make_async_copy(ew_hbm.at[pl.ds(off, K)], w_v[b],
                                  isem[b]).wait()
            if gather:
                pltpu.async_copy(y_hbm.at[s_idx[b]], rows[b], gsem[b])

        def wait_gather(b):
            if gather:
                pltpu.make_async_copy(y_hbm.at[s_idx[b]], rows[b],
                                      gsem[b]).wait()

        def scale(b):
            def grp(g, _):
                w16 = w_v[b][pl.ds(g * LANES, LANES)]
                for l in range(LANES):
                    k = g * LANES + l
                    w = jnp.full((LANES,), w16[l], jnp.float32)
                    if gather:
                        for f in range(nf):
                            sl = pl.ds(f * LANES, LANES)
                            rows[b][k, sl] = rows[b][k, sl] * w
                    else:
                        rows[b][k, pl.ds(0, LANES)] = w
                return _

            lax.fori_loop(0, K // LANES, grp, None)

        def start_scatter(b):
            pltpu.async_copy(rows[b], acc_sh.at[d_idx[b]], ssem[b], add=True)

        def wait_scatter(b):
            pltpu.make_async_copy(rows[b], acc_sh.at[d_idx[b]], ssem[b]).wait()

        for b in range(NB):
            fetch(b, b)

        def body(jj, _):
            for b in range(NB):
                wait_gather(b)
                scale(b)
                start_scatter(b)
            for b in range(NB):
                wait_scatter(b)
                fetch(b, jj * NB + b)
            return _

        lax.fori_loop(1, nj, body, None)

        for b in range(NB):
            wait_gather(b)
            scale(b)
            start_scatter(b)
        for b in range(NB):
            wait_scatter(b)

        plsc.subcore_barrier()
        for t in range(Np // NS // K):
            r0 = sid * (Np // NS) + t * K
            pltpu.sync_copy(acc_sh.at[pl.ds(r0, K)], out_hbm.at[cid, pl.ds(r0, K)])

    return agg_kernel


def _dinv_from_deg(dp):
    deg = jnp.sum(dp, axis=0) + 1.0
    return jnp.where(deg > 0, lax.rsqrt(jnp.maximum(deg, 1e-12)), 0.0)


def _tc_first(x_ref, w_ref, dp_ref, y_ref):
    dinv = _dinv_from_deg(dp_ref[...])
    xw = jnp.dot(x_ref[...], w_ref[...], preferred_element_type=jnp.float32)
    y_ref[...] = dinv[:, None] * xw


def _tc_mid(a0_ref, a1_ref, y1_ref, dp_ref, b_ref, w_ref, y2_ref):
    dinv = _dinv_from_deg(dp_ref[...])
    pre = dinv[:, None] * (a0_ref[...] + a1_ref[...] + y1_ref[...]) + b_ref[...]
    h = jnp.where(pre >= 0, pre, 0.01 * pre)
    hw = jnp.dot(h, w_ref[...], preferred_element_type=jnp.float32)
    y2_ref[...] = dinv[:, None] * hw


def _tc_last(a0_ref, a1_ref, y2_ref, dp_ref, b_ref, out_ref):
    dinv = _dinv_from_deg(dp_ref[...])
    out_ref[...] = dinv[:, None] * (a0_ref[...] + a1_ref[...] + y2_ref[...]) + b_ref[...]


def _row_spec(F):
    return pl.BlockSpec((BM, F), lambda i: (i, 0))


def _tc_call(body, Np, F, specs, n_rows=None):
    return pl.pallas_call(
        body,
        grid=(Np // BM,),
        in_specs=specs,
        out_specs=_row_spec(F),
        out_shape=jax.ShapeDtypeStruct((n_rows or Np, F), jnp.float32),
    )


def kernel(x, edge_index, edge_weight, W1, b1, W2, b2):
    N, F = x.shape
    E = edge_index.shape[1]
    Np = _cdiv(N, NS * K) * NS * K
    Ep = _cdiv(E, NC * NS * K * NB) * NC * NS * K * NB

    src = edge_index[0]
    dst = edge_index[1]
    if Ep != E:
        pad = Ep - E
        spread = (jnp.arange(pad, dtype=jnp.int32) * 97) % N
        src = jnp.concatenate([src, spread])
        dst = jnp.concatenate([dst, spread])
        edge_weight = jnp.pad(edge_weight, (0, pad))
    xp = jnp.pad(x, ((0, Np - N), (0, 0))) if Np != N else x
    b1r = b1.reshape(1, F)
    b2r = b2.reshape(1, F)

    deg_k = _make_deg_kernel(Np, Ep)
    agg_k = _make_agg_kernel(Np, Ep, F, gather=True)

    degp = deg_k(dst, edge_weight)

    dp_spec = pl.BlockSpec((NC * NS, BM), lambda i: (0, i))
    w_spec = pl.BlockSpec((F, F), lambda i: (0, 0))
    b_spec = pl.BlockSpec((1, F), lambda i: (0, 0))
    row = _row_spec(F)

    y1 = _tc_call(_tc_first, Np, F, [row, w_spec, dp_spec])(xp, W1, degp)

    acc1 = agg_k(y1, src, dst, edge_weight)

    y2 = _tc_call(_tc_mid, Np, F, [row, row, row, dp_spec, b_spec, w_spec])(
        acc1[0], acc1[1], y1, degp, b1r, W2)

    acc2 = agg_k(y2, src, dst, edge_weight)

    out = _tc_call(_tc_last, Np, F, [row, row, row, dp_spec, b_spec],
                   n_rows=N)(acc2[0], acc2[1], y2, degp, b2r)

    return out
